# baseline (speedup 1.0000x reference)
import functools

import jax
import jax.numpy as jnp
from jax import lax
from jax.experimental import pallas as pl
from jax.experimental.pallas import tpu as pltpu
from jax.experimental.pallas import tpu_sc as plsc

BATCH = 16384
NUM_FIELDS = 26
EMBED_DIM = 64
FIELD_SIZE = 100000
TOTAL = BATCH * NUM_FIELDS

_info = plsc.get_sparse_core_info()
NC = _info.num_cores
NS = _info.num_subcores
LANES = _info.num_lanes
NW = NC * NS
ROWS_PER_W = TOTAL // NW

CHUNK = 512
GATHERS = CHUNK // 128
NITER = ROWS_PER_W // CHUNK


def _sc_gather(x_flat, table):
    mesh = plsc.VectorSubcoreMesh(core_axis_name="c", subcore_axis_name="s")

    @functools.partial(
        pl.kernel,
        mesh=mesh,
        compiler_params=pltpu.CompilerParams(use_tc_tiling_on_sc=False),
        out_type=jax.ShapeDtypeStruct((TOTAL, EMBED_DIM), jnp.float32),
        scratch_types=[
            pltpu.VMEM((CHUNK,), jnp.int32),
            pltpu.VMEM((GATHERS, 128), jnp.int32),
            pltpu.VMEM((CHUNK, EMBED_DIM), jnp.float32),
            pltpu.SemaphoreType.DMA,
        ],
    )
    def k(x_hbm, table_hbm, out_hbm, xbuf, idxbuf, rowbuf, sem):
        wid = lax.axis_index("s") * NC + lax.axis_index("c")
        wbase = wid * ROWS_PER_W
        lane = lax.iota(jnp.int32, LANES)

        def body(it, carry):
            base = wbase + it * CHUNK
            pltpu.sync_copy(x_hbm.at[pl.ds(base, CHUNK)], xbuf)
            for j in range(CHUNK // LANES):
                pos = base + j * LANES + lane
                off = lax.rem(pos, NUM_FIELDS) * FIELD_SIZE
                xv = xbuf[pl.ds(j * LANES, LANES)]
                row = (j * LANES) // 128
                col = (j * LANES) % 128
                idxbuf[row, pl.ds(col, LANES)] = xv + off
            copies = [
                pltpu.async_copy(
                    table_hbm.at[idxbuf.at[t]],
                    rowbuf.at[pl.ds(t * 128, 128)],
                    sem,
                )
                for t in range(GATHERS)
            ]
            for c in copies:
                c.wait()
            pltpu.sync_copy(rowbuf, out_hbm.at[pl.ds(base, CHUNK)])
            return carry

        lax.fori_loop(0, NITER, body, 0)

    return k(x_flat, table)


def kernel(x, table):
    x_flat = x.reshape(-1)
    out = _sc_gather(x_flat, table)
    return out.reshape(BATCH, NUM_FIELDS, EMBED_DIM)


# speedup vs baseline: 1.0100x; 1.0100x over previous
import functools

import jax
import jax.numpy as jnp
from jax import lax
from jax.experimental import pallas as pl
from jax.experimental.pallas import tpu as pltpu
from jax.experimental.pallas import tpu_sc as plsc

BATCH = 16384
NUM_FIELDS = 26
EMBED_DIM = 64
FIELD_SIZE = 100000
TOTAL = BATCH * NUM_FIELDS

_info = plsc.get_sparse_core_info()
NC = _info.num_cores
NS = _info.num_subcores
LANES = _info.num_lanes
NW = NC * NS
BROWS_PER_W = BATCH // NW
ROWS_PER_W = TOTAL // NW

G_IDX = 128
N_GROUPS = ROWS_PER_W // G_IDX
CHUNK = 4 * G_IDX
NCHUNK = ROWS_PER_W // CHUNK


def _sc_gather(x, table):
    mesh = plsc.VectorSubcoreMesh(core_axis_name="c", subcore_axis_name="s")

    @functools.partial(
        pl.kernel,
        mesh=mesh,
        compiler_params=pltpu.CompilerParams(
            use_tc_tiling_on_sc=False, needs_layout_passes=False
        ),
        out_type=jax.ShapeDtypeStruct((TOTAL, EMBED_DIM), jnp.float32),
        scratch_types=[
            pltpu.VMEM((BROWS_PER_W, NUM_FIELDS), jnp.int32),
            pltpu.VMEM((N_GROUPS, G_IDX), jnp.int32),
            pltpu.VMEM((2, CHUNK, EMBED_DIM), jnp.float32),
            pltpu.SemaphoreType.DMA,
        ],
    )
    def k(x_hbm, table_hbm, out_hbm, xbuf, idxbuf, rowbuf, sem):
        wid = lax.axis_index("s") * NC + lax.axis_index("c")
        lane = lax.iota(jnp.int32, LANES)

        pltpu.sync_copy(x_hbm.at[pl.ds(wid * BROWS_PER_W, BROWS_PER_W)], xbuf)

        inv13 = jnp.int32(-991146299)

        def idx_body(grp, carry):
            for j in range(G_IDX // LANES):
                p = grp * G_IDX + j * LANES + lane
                c = lax.rem(p, NUM_FIELDS)
                r = lax.shift_right_logical(p - c, 1) * inv13
                v = plsc.load_gather(xbuf, [r, c])
                idx = v + c * FIELD_SIZE
                idx = lax.max(lax.min(idx, NUM_FIELDS * FIELD_SIZE - 1), 0)
                idxbuf[grp, pl.ds(j * LANES, LANES)] = idx
            return carry

        lax.fori_loop(0, N_GROUPS, idx_body, 0)

        out_base = wid * ROWS_PER_W

        def fire(g):
            buf = g % 2
            return [
                pltpu.async_copy(
                    table_hbm.at[idxbuf.at[4 * g + t]],
                    rowbuf.at[buf, pl.ds(t * G_IDX, G_IDX)],
                    sem,
                )
                for t in range(4)
            ]

        def drain(g, copies):
            for c in copies:
                c.wait()
            pltpu.sync_copy(
                rowbuf.at[g % 2],
                out_hbm.at[pl.ds(out_base + g * CHUNK, CHUNK)],
            )

        prev = fire(0)
        for g in range(1, NCHUNK):
            cur = fire(g)
            drain(g - 1, prev)
            prev = cur
        drain(NCHUNK - 1, prev)

    return k(x, table)


def kernel(x, table):
    out = _sc_gather(x, table)
    return out.reshape(BATCH, NUM_FIELDS, EMBED_DIM)
